# baseline (speedup 1.0000x reference)
import functools

import jax
import jax.numpy as jnp
from jax import lax
from jax.experimental import pallas as pl
from jax.experimental.pallas import tpu as pltpu
from jax.experimental.pallas import tpu_sc as plsc

_N_E = 8192
_E_DIM = 64
_BETA = 0.25

_M_BLK = 1024
_E_CHUNK = 4096


def _argmin_body(zf_ref, cb_ref, idx_ref):
    zblk = zf_ref[...]
    zsq = jnp.sum(zblk * zblk, axis=1, keepdims=True)
    zm2 = zblk * (-2.0)

    bmin = None
    for c in range(_N_E // _E_CHUNK):
        cb = cb_ref[pl.ds(c * _E_CHUNK, _E_CHUNK), :]
        e2 = jnp.sum(cb * cb, axis=1)
        mm2 = lax.dot_general(zm2, cb, (((1,), (1,)), ((), ())),
                              preferred_element_type=jnp.float32)
        d = (zsq + e2[None, :]) + mm2
        cmin = jnp.min(d, axis=1, keepdims=True)
        ids = lax.broadcasted_iota(jnp.int32, d.shape, 1)
        cidx = jnp.min(jnp.where(d == cmin, ids, _N_E),
                       axis=1, keepdims=True) + c * _E_CHUNK
        if bmin is None:
            bmin = cmin.astype(jnp.bfloat16).astype(jnp.float32)
            bidx = cidx
        else:
            take = cmin < bmin
            cmin_b = cmin.astype(jnp.bfloat16).astype(jnp.float32)
            bmin = jnp.where(take, cmin_b, bmin)
            bidx = jnp.where(take, cidx, bidx)

    idx_ref[...] = bidx


def _argmin_call(zf, codebook):
    n = zf.shape[0]
    return pl.pallas_call(
        _argmin_body,
        grid=(n // _M_BLK,),
        in_specs=[
            pl.BlockSpec((_M_BLK, _E_DIM), lambda i: (i, 0)),
            pl.BlockSpec((_N_E, _E_DIM), lambda i: (0, 0)),
        ],
        out_specs=pl.BlockSpec((_M_BLK, 1), lambda i: (i, 0)),
        out_shape=jax.ShapeDtypeStruct((n, 1), jnp.int32),
    )(zf, codebook)


def _gather_call(cb_pad, idx):
    n = idx.shape[0]
    row = cb_pad.shape[1]
    info = plsc.get_sparse_core_info()
    nw = info.num_cores * info.num_subcores
    b_per_w = n // nw
    mesh = plsc.VectorSubcoreMesh(core_axis_name="c", subcore_axis_name="s")

    @functools.partial(
        pl.kernel, mesh=mesh,
        out_type=jax.ShapeDtypeStruct((n, row), jnp.float32),
        scratch_types=[
            pltpu.VMEM((b_per_w,), jnp.int32),
            pltpu.VMEM((b_per_w, row), jnp.float32),
            pltpu.SemaphoreType.DMA,
        ],
    )
    def gather_k(cb_hbm, idx_hbm, out_hbm, idx_v, rows_v, sem):
        wid = lax.axis_index("s") * info.num_cores + lax.axis_index("c")
        base = wid * b_per_w
        pltpu.sync_copy(idx_hbm.at[pl.ds(base, b_per_w)], idx_v)
        pltpu.async_copy(cb_hbm.at[idx_v], rows_v, sem).wait()
        pltpu.sync_copy(rows_v, out_hbm.at[pl.ds(base, b_per_w)])

    return gather_k(cb_pad, idx)


def _finish_body(zf_ref, zq_ref, st_ref, loss_ref):
    zp = zf_ref[...]
    zq = zq_ref[:, :_E_DIM]
    diff = zq - zp
    st_ref[...] = zp + diff
    m = jnp.mean(diff * diff)
    loss_ref[0, 0] = m + _BETA * m


def _finish_call(zf, zq_pad):
    n = zf.shape[0]
    return pl.pallas_call(
        _finish_body,
        in_specs=[
            pl.BlockSpec((n, _E_DIM), lambda: (0, 0)),
            pl.BlockSpec((n, 2 * _E_DIM), lambda: (0, 0)),
        ],
        out_specs=[
            pl.BlockSpec((n, _E_DIM), lambda: (0, 0)),
            pl.BlockSpec(memory_space=pltpu.SMEM),
        ],
        out_shape=[
            jax.ShapeDtypeStruct((n, _E_DIM), jnp.float32),
            jax.ShapeDtypeStruct((1, 1), jnp.float32),
        ],
    )(zf, zq_pad)


def kernel(z, codebook):
    b, c, d, h, w = z.shape
    zf = jnp.transpose(z, (0, 2, 3, 4, 1)).reshape(-1, c)
    idx = _argmin_call(zf, codebook).reshape(-1)
    cb_pad = jnp.pad(codebook, ((0, 0), (0, _E_DIM)))
    zq_pad = _gather_call(cb_pad, idx)
    st_flat, loss11 = _finish_call(zf, zq_pad)
    z_q = st_flat.reshape(b, d, h, w, c).transpose(0, 4, 1, 2, 3)
    return z_q, loss11[0, 0], idx


# speedup vs baseline: 1.2235x; 1.2235x over previous
import functools

import jax
import jax.numpy as jnp
from jax import lax
from jax.experimental import pallas as pl
from jax.experimental.pallas import tpu as pltpu
from jax.experimental.pallas import tpu_sc as plsc

_N_E = 8192
_E_DIM = 64
_BETA = 0.25

_M_BLK = 1024
_E_CHUNK = 4096


def _argmin_body(zf_ref, cb_ref, idx_ref):
    zblk = zf_ref[...]
    zsq = jnp.sum(zblk * zblk, axis=1, keepdims=True)
    zm2 = zblk * (-2.0)

    nl = 128
    lane = lax.broadcasted_iota(jnp.int32, (_M_BLK, nl), 1)

    bmin = None
    for c in range(_N_E // _E_CHUNK):
        cb = cb_ref[pl.ds(c * _E_CHUNK, _E_CHUNK), :]
        e2 = jnp.sum(cb * cb, axis=1)
        mm2 = lax.dot_general(zm2, cb, (((1,), (1,)), ((), ())),
                              preferred_element_type=jnp.float32)
        m = (zsq + e2[None, :nl]) + mm2[:, :nl]
        ci = jnp.zeros((_M_BLK, nl), jnp.int32)
        for cc in range(1, _E_CHUNK // nl):
            lo, hi = cc * nl, (cc + 1) * nl
            dc = (zsq + e2[None, lo:hi]) + mm2[:, lo:hi]
            lt = dc < m
            m = jnp.where(lt, dc, m)
            ci = jnp.where(lt, jnp.int32(cc), ci)
        cmin = jnp.min(m, axis=1, keepdims=True)
        j = ci * nl + lane
        cidx = jnp.min(jnp.where(m == cmin, j, _N_E),
                       axis=1, keepdims=True) + c * _E_CHUNK
        if bmin is None:
            bmin = cmin.astype(jnp.bfloat16).astype(jnp.float32)
            bidx = cidx
        else:
            take = cmin < bmin
            cmin_b = cmin.astype(jnp.bfloat16).astype(jnp.float32)
            bmin = jnp.where(take, cmin_b, bmin)
            bidx = jnp.where(take, cidx, bidx)

    idx_ref[...] = bidx


def _argmin_call(zf, codebook):
    n = zf.shape[0]
    return pl.pallas_call(
        _argmin_body,
        grid=(n // _M_BLK,),
        in_specs=[
            pl.BlockSpec((_M_BLK, _E_DIM), lambda i: (i, 0)),
            pl.BlockSpec((_N_E, _E_DIM), lambda i: (0, 0)),
        ],
        out_specs=pl.BlockSpec((_M_BLK, 1), lambda i: (i, 0)),
        out_shape=jax.ShapeDtypeStruct((n, 1), jnp.int32),
    )(zf, codebook)


def _gather_call(cb_pad, idx):
    n = idx.shape[0]
    row = cb_pad.shape[1]
    info = plsc.get_sparse_core_info()
    nw = info.num_cores * info.num_subcores
    b_per_w = n // nw
    mesh = plsc.VectorSubcoreMesh(core_axis_name="c", subcore_axis_name="s")

    @functools.partial(
        pl.kernel, mesh=mesh,
        out_type=jax.ShapeDtypeStruct((n, row), jnp.float32),
        scratch_types=[
            pltpu.VMEM((b_per_w,), jnp.int32),
            pltpu.VMEM((b_per_w, row), jnp.float32),
            pltpu.SemaphoreType.DMA,
        ],
    )
    def gather_k(cb_hbm, idx_hbm, out_hbm, idx_v, rows_v, sem):
        wid = lax.axis_index("s") * info.num_cores + lax.axis_index("c")
        base = wid * b_per_w
        pltpu.sync_copy(idx_hbm.at[pl.ds(base, b_per_w)], idx_v)
        pltpu.async_copy(cb_hbm.at[idx_v], rows_v, sem).wait()
        pltpu.sync_copy(rows_v, out_hbm.at[pl.ds(base, b_per_w)])

    return gather_k(cb_pad, idx)


def _finish_body(zf_ref, zq_ref, st_ref, loss_ref):
    zp = zf_ref[...]
    zq = zq_ref[:, :_E_DIM]
    diff = zq - zp
    st_ref[...] = zp + diff
    m = jnp.mean(diff * diff)
    loss_ref[0, 0] = m + _BETA * m


def _finish_call(zf, zq_pad):
    n = zf.shape[0]
    return pl.pallas_call(
        _finish_body,
        in_specs=[
            pl.BlockSpec((n, _E_DIM), lambda: (0, 0)),
            pl.BlockSpec((n, 2 * _E_DIM), lambda: (0, 0)),
        ],
        out_specs=[
            pl.BlockSpec((n, _E_DIM), lambda: (0, 0)),
            pl.BlockSpec(memory_space=pltpu.SMEM),
        ],
        out_shape=[
            jax.ShapeDtypeStruct((n, _E_DIM), jnp.float32),
            jax.ShapeDtypeStruct((1, 1), jnp.float32),
        ],
    )(zf, zq_pad)


def kernel(z, codebook):
    b, c, d, h, w = z.shape
    zf = jnp.transpose(z, (0, 2, 3, 4, 1)).reshape(-1, c)
    idx = _argmin_call(zf, codebook).reshape(-1)
    cb_pad = jnp.pad(codebook, ((0, 0), (0, _E_DIM)))
    zq_pad = _gather_call(cb_pad, idx)
    st_flat, loss11 = _finish_call(zf, zq_pad)
    z_q = st_flat.reshape(b, d, h, w, c).transpose(0, 4, 1, 2, 3)
    return z_q, loss11[0, 0], idx


# speedup vs baseline: 1.2312x; 1.0063x over previous
import functools

import jax
import jax.numpy as jnp
from jax import lax
from jax.experimental import pallas as pl
from jax.experimental.pallas import tpu as pltpu
from jax.experimental.pallas import tpu_sc as plsc

_N_E = 8192
_E_DIM = 64
_BETA = 0.25

_M_BLK = 1024
_E_CHUNK = 4096


def _argmin_body(zf_ref, cb_ref, idx_ref):
    nl = 128
    lane = lax.broadcasted_iota(jnp.int32, (_M_BLK, nl), 1)
    n_rows = zf_ref.shape[0]

    for r in range(n_rows // _M_BLK):
        zblk = zf_ref[pl.ds(r * _M_BLK, _M_BLK), :]
        zsq = jnp.sum(zblk * zblk, axis=1, keepdims=True)
        zm2 = zblk * (-2.0)

        bmin = None
        for c in range(_N_E // _E_CHUNK):
            cb = cb_ref[pl.ds(c * _E_CHUNK, _E_CHUNK), :]
            e2 = jnp.sum(cb * cb, axis=1)
            mm2 = lax.dot_general(zm2, cb, (((1,), (1,)), ((), ())),
                                  preferred_element_type=jnp.float32)
            m = (zsq + e2[None, :nl]) + mm2[:, :nl]
            ci = jnp.zeros((_M_BLK, nl), jnp.int32)
            for cc in range(1, _E_CHUNK // nl):
                lo, hi = cc * nl, (cc + 1) * nl
                dc = (zsq + e2[None, lo:hi]) + mm2[:, lo:hi]
                lt = dc < m
                m = jnp.where(lt, dc, m)
                ci = jnp.where(lt, jnp.int32(cc), ci)
            cmin = jnp.min(m, axis=1, keepdims=True)
            j = ci * nl + lane
            cidx = jnp.min(jnp.where(m == cmin, j, _N_E),
                           axis=1, keepdims=True) + c * _E_CHUNK
            if bmin is None:
                bmin = cmin.astype(jnp.bfloat16).astype(jnp.float32)
                bidx = cidx
            else:
                take = cmin < bmin
                cmin_b = cmin.astype(jnp.bfloat16).astype(jnp.float32)
                bmin = jnp.where(take, cmin_b, bmin)
                bidx = jnp.where(take, cidx, bidx)

        idx_ref[pl.ds(r * _M_BLK, _M_BLK), :] = bidx


def _argmin_call(zf, codebook):
    n = zf.shape[0]
    return pl.pallas_call(
        _argmin_body,
        in_specs=[
            pl.BlockSpec((n, _E_DIM), lambda: (0, 0)),
            pl.BlockSpec((_N_E, _E_DIM), lambda: (0, 0)),
        ],
        out_specs=pl.BlockSpec((n, 1), lambda: (0, 0)),
        out_shape=jax.ShapeDtypeStruct((n, 1), jnp.int32),
    )(zf, codebook)


def _gather_call(cb_pad, idx):
    n = idx.shape[0]
    row = cb_pad.shape[1]
    info = plsc.get_sparse_core_info()
    nw = info.num_cores * info.num_subcores
    b_per_w = n // nw
    mesh = plsc.VectorSubcoreMesh(core_axis_name="c", subcore_axis_name="s")

    @functools.partial(
        pl.kernel, mesh=mesh,
        out_type=jax.ShapeDtypeStruct((n, row), jnp.float32),
        scratch_types=[
            pltpu.VMEM((b_per_w,), jnp.int32),
            pltpu.VMEM((b_per_w, row), jnp.float32),
            pltpu.SemaphoreType.DMA,
        ],
    )
    def gather_k(cb_hbm, idx_hbm, out_hbm, idx_v, rows_v, sem):
        wid = lax.axis_index("s") * info.num_cores + lax.axis_index("c")
        base = wid * b_per_w
        pltpu.sync_copy(idx_hbm.at[pl.ds(base, b_per_w)], idx_v)
        pltpu.async_copy(cb_hbm.at[idx_v], rows_v, sem).wait()
        pltpu.sync_copy(rows_v, out_hbm.at[pl.ds(base, b_per_w)])

    return gather_k(cb_pad, idx)


def _finish_body(zf_ref, zq_ref, st_ref, loss_ref):
    zp = zf_ref[...]
    zq = zq_ref[:, :_E_DIM]
    diff = zq - zp
    st_ref[...] = zp + diff
    m = jnp.mean(diff * diff)
    loss_ref[0, 0] = m + _BETA * m


def _finish_call(zf, zq_pad):
    n = zf.shape[0]
    return pl.pallas_call(
        _finish_body,
        in_specs=[
            pl.BlockSpec((n, _E_DIM), lambda: (0, 0)),
            pl.BlockSpec((n, 2 * _E_DIM), lambda: (0, 0)),
        ],
        out_specs=[
            pl.BlockSpec((n, _E_DIM), lambda: (0, 0)),
            pl.BlockSpec(memory_space=pltpu.SMEM),
        ],
        out_shape=[
            jax.ShapeDtypeStruct((n, _E_DIM), jnp.float32),
            jax.ShapeDtypeStruct((1, 1), jnp.float32),
        ],
    )(zf, zq_pad)


def kernel(z, codebook):
    b, c, d, h, w = z.shape
    zf = jnp.transpose(z, (0, 2, 3, 4, 1)).reshape(-1, c)
    idx = _argmin_call(zf, codebook).reshape(-1)
    cb_pad = jnp.pad(codebook, ((0, 0), (0, _E_DIM)))
    zq_pad = _gather_call(cb_pad, idx)
    st_flat, loss11 = _finish_call(zf, zq_pad)
    z_q = st_flat.reshape(b, d, h, w, c).transpose(0, 4, 1, 2, 3)
    return z_q, loss11[0, 0], idx


# speedup vs baseline: 1.2442x; 1.0106x over previous
import functools

import jax
import jax.numpy as jnp
from jax import lax
from jax.experimental import pallas as pl
from jax.experimental.pallas import tpu as pltpu
from jax.experimental.pallas import tpu_sc as plsc

_N_E = 8192
_E_DIM = 64
_BETA = 0.25

_M_BLK = 1024
_E_CHUNK = 4096


def _argmin_body(zf_ref, cb_ref, idx_ref, cbp_ref):
    nl = 128
    lane = lax.broadcasted_iota(jnp.int32, (_M_BLK, nl), 1)
    n_rows = zf_ref.shape[0]

    cbp_ref[:, :_E_DIM] = cb_ref[...]
    cbp_ref[:, _E_DIM:] = jnp.zeros((_N_E, _E_DIM), jnp.float32)

    for r in range(n_rows // _M_BLK):
        zblk = zf_ref[pl.ds(r * _M_BLK, _M_BLK), :]
        zsq = jnp.sum(zblk * zblk, axis=1, keepdims=True)
        zm2 = zblk * (-2.0)

        bmin = None
        for c in range(_N_E // _E_CHUNK):
            cb = cb_ref[pl.ds(c * _E_CHUNK, _E_CHUNK), :]
            e2 = jnp.sum(cb * cb, axis=1)
            mm2 = lax.dot_general(zm2, cb, (((1,), (1,)), ((), ())),
                                  preferred_element_type=jnp.float32)
            m = (zsq + e2[None, :nl]) + mm2[:, :nl]
            ci = jnp.zeros((_M_BLK, nl), jnp.int32)
            for cc in range(1, _E_CHUNK // nl):
                lo, hi = cc * nl, (cc + 1) * nl
                dc = (zsq + e2[None, lo:hi]) + mm2[:, lo:hi]
                lt = dc < m
                m = jnp.where(lt, dc, m)
                ci = jnp.where(lt, jnp.int32(cc), ci)
            cmin = jnp.min(m, axis=1, keepdims=True)
            j = ci * nl + lane
            cidx = jnp.min(jnp.where(m == cmin, j, _N_E),
                           axis=1, keepdims=True) + c * _E_CHUNK
            if bmin is None:
                bmin = cmin.astype(jnp.bfloat16).astype(jnp.float32)
                bidx = cidx
            else:
                take = cmin < bmin
                cmin_b = cmin.astype(jnp.bfloat16).astype(jnp.float32)
                bmin = jnp.where(take, cmin_b, bmin)
                bidx = jnp.where(take, cidx, bidx)

        idx_ref[pl.ds(r * _M_BLK, _M_BLK), :] = bidx


def _argmin_call(zf, codebook):
    n = zf.shape[0]
    return pl.pallas_call(
        _argmin_body,
        in_specs=[
            pl.BlockSpec((n, _E_DIM), lambda: (0, 0)),
            pl.BlockSpec((_N_E, _E_DIM), lambda: (0, 0)),
        ],
        out_specs=[
            pl.BlockSpec((n, 1), lambda: (0, 0)),
            pl.BlockSpec((_N_E, 2 * _E_DIM), lambda: (0, 0)),
        ],
        out_shape=[
            jax.ShapeDtypeStruct((n, 1), jnp.int32),
            jax.ShapeDtypeStruct((_N_E, 2 * _E_DIM), jnp.float32),
        ],
    )(zf, codebook)


def _gather_call(cb_pad, idx):
    n = idx.shape[0]
    row = cb_pad.shape[1]
    info = plsc.get_sparse_core_info()
    nw = info.num_cores * info.num_subcores
    b_per_w = n // nw
    mesh = plsc.VectorSubcoreMesh(core_axis_name="c", subcore_axis_name="s")

    @functools.partial(
        pl.kernel, mesh=mesh,
        out_type=jax.ShapeDtypeStruct((n, row), jnp.float32),
        scratch_types=[
            pltpu.VMEM((b_per_w,), jnp.int32),
            pltpu.VMEM((b_per_w, row), jnp.float32),
            pltpu.SemaphoreType.DMA,
        ],
    )
    def gather_k(cb_hbm, idx_hbm, out_hbm, idx_v, rows_v, sem):
        wid = lax.axis_index("s") * info.num_cores + lax.axis_index("c")
        base = wid * b_per_w
        pltpu.sync_copy(idx_hbm.at[pl.ds(base, b_per_w)], idx_v)
        pltpu.async_copy(cb_hbm.at[idx_v], rows_v, sem).wait()
        pltpu.sync_copy(rows_v, out_hbm.at[pl.ds(base, b_per_w)])

    return gather_k(cb_pad, idx)


def _finish_body(zf_ref, zq_ref, st_ref, loss_ref):
    zp = zf_ref[...]
    zq = zq_ref[:, :_E_DIM]
    diff = zq - zp
    st_ref[...] = zp + diff
    m = jnp.mean(diff * diff)
    loss_ref[0, 0] = m + _BETA * m


def _finish_call(zf, zq_pad):
    n = zf.shape[0]
    return pl.pallas_call(
        _finish_body,
        in_specs=[
            pl.BlockSpec((n, _E_DIM), lambda: (0, 0)),
            pl.BlockSpec((n, 2 * _E_DIM), lambda: (0, 0)),
        ],
        out_specs=[
            pl.BlockSpec((n, _E_DIM), lambda: (0, 0)),
            pl.BlockSpec(memory_space=pltpu.SMEM),
        ],
        out_shape=[
            jax.ShapeDtypeStruct((n, _E_DIM), jnp.float32),
            jax.ShapeDtypeStruct((1, 1), jnp.float32),
        ],
    )(zf, zq_pad)


def kernel(z, codebook):
    b, c, d, h, w = z.shape
    zf = jnp.transpose(z, (0, 2, 3, 4, 1)).reshape(-1, c)
    idx2, cb_pad = _argmin_call(zf, codebook)
    idx = idx2.reshape(-1)
    zq_pad = _gather_call(cb_pad, idx)
    st_flat, loss11 = _finish_call(zf, zq_pad)
    z_q = st_flat.reshape(b, d, h, w, c).transpose(0, 4, 1, 2, 3)
    return z_q, loss11[0, 0], idx
